# baseline (speedup 1.0000x reference)
import jax
import jax.numpy as jnp
from jax import lax
from jax.experimental import pallas as pl
from jax.experimental.pallas import tpu as pltpu
from jax.experimental.pallas import tpu_sc as plsc

VOCAB = 100000
EMB_DIM = 128
BATCH = 4096
HIST = 50

NUM_CORES = 2
NUM_SUBCORES = 16
NUM_WORKERS = NUM_CORES * NUM_SUBCORES
TOTAL_ROWS = BATCH * HIST
ROWS_PER_WORKER = TOTAL_ROWS // NUM_WORKERS
CHUNK = 128
SUPER = 2
NSUP = ROWS_PER_WORKER // (CHUNK * SUPER)
NCHUNKS_TOTAL = TOTAL_ROWS // CHUNK
NBUF = 3
LOOK = NBUF - 1


def _emb_body(idx_hbm, table_hbm, out_hbm, idx_v, *bufs_and_sems):
    bufs = bufs_and_sems[:NBUF]
    gsems = bufs_and_sems[NBUF:2 * NBUF]
    ssems = bufs_and_sems[2 * NBUF:3 * NBUF]

    wid = lax.axis_index("s") * NUM_CORES + lax.axis_index("c")
    pltpu.sync_copy(idx_hbm.at[wid], idx_v)
    base = wid * ROWS_PER_WORKER

    def fire_gather(c, b):
        for k in range(SUPER):
            pltpu.async_copy(
                table_hbm.at[idx_v.at[c * SUPER + k]],
                bufs[b].at[pl.ds(k * CHUNK, CHUNK)], gsems[b])

    def wait_gather(c, b):
        for k in range(SUPER):
            pltpu.make_async_copy(
                table_hbm.at[idx_v.at[0]],
                bufs[b].at[pl.ds(k * CHUNK, CHUNK)], gsems[b]).wait()

    def fire_scatter(c, b):
        pltpu.async_copy(
            bufs[b], out_hbm.at[pl.ds(base + c * SUPER * CHUNK, SUPER * CHUNK)],
            ssems[b])

    def wait_scatter(b):
        pltpu.make_async_copy(
            bufs[b], out_hbm.at[pl.ds(base, SUPER * CHUNK)], ssems[b]).wait()

    for b in range(LOOK):
        fire_gather(b, b)

    wait_gather(0, 0)
    fire_scatter(0, 0)
    fire_gather(LOOK, LOOK % NBUF)

    def step(c, b, tb):
        wait_gather(c, b)
        fire_scatter(c, b)
        wait_scatter(tb)
        fire_gather(c + LOOK, tb)

    nsteady = NSUP - LOOK - 1
    ngroups = nsteady // NBUF
    nrem = nsteady % NBUF

    def outer(g, carry):
        for bp in range(NBUF):
            c = g * NBUF + 1 + bp
            step(c, (bp + 1) % NBUF, (1 + bp + LOOK) % NBUF)
        return carry

    lax.fori_loop(0, ngroups, outer, 0)
    for r in range(nrem):
        c = ngroups * NBUF + 1 + r
        step(c, c % NBUF, (c + LOOK) % NBUF)

    for c in range(NSUP - LOOK, NSUP):
        b = c % NBUF
        wait_gather(c, b)
        fire_scatter(c, b)
    for b in range(NBUF):
        wait_scatter(b)


@jax.jit
def _emb_call(idx, weight):
    mesh = plsc.VectorSubcoreMesh(
        core_axis_name="c", subcore_axis_name="s",
        num_cores=NUM_CORES, num_subcores=NUM_SUBCORES,
    )
    run = pl.kernel(
        _emb_body,
        out_type=jax.ShapeDtypeStruct((TOTAL_ROWS, EMB_DIM), jnp.float32),
        mesh=mesh,
        scratch_types=(
            [pltpu.VMEM((NSUP * SUPER, CHUNK), jnp.int32)]
            + [pltpu.VMEM((SUPER * CHUNK, EMB_DIM), jnp.float32) for _ in range(NBUF)]
            + [pltpu.SemaphoreType.DMA for _ in range(2 * NBUF)]
        ),
    )
    return run(idx, weight)


def kernel(input, weight):
    idx = input.astype(jnp.int32).reshape(NUM_WORKERS, NSUP * SUPER, CHUNK)
    out = _emb_call(idx, weight)
    return out.reshape(BATCH, HIST, EMB_DIM)


# speedup vs baseline: 1.7870x; 1.7870x over previous
import jax
import jax.numpy as jnp
from jax import lax
from jax.experimental import pallas as pl
from jax.experimental.pallas import tpu as pltpu
from jax.experimental.pallas import tpu_sc as plsc

VOCAB = 100000
EMB_DIM = 128
BATCH = 4096
HIST = 50

NUM_CORES = 2
NUM_SUBCORES = 16
NUM_WORKERS = NUM_CORES * NUM_SUBCORES
BATCH_PER_WORKER = BATCH // NUM_WORKERS
SUPER = 4
NSUP = BATCH_PER_WORKER // SUPER
NBUF = 4
LOOK = NBUF - 1


def _emb_body(idx_hbm, table_hbm, out_hbm, idx_v, *bufs_and_sems):
    bufs = bufs_and_sems[:NBUF]
    gsems = bufs_and_sems[NBUF:2 * NBUF]
    ssems = bufs_and_sems[2 * NBUF:3 * NBUF]

    wid = lax.axis_index("s") * NUM_CORES + lax.axis_index("c")
    pltpu.sync_copy(idx_hbm.at[wid], idx_v)
    base = wid * BATCH_PER_WORKER

    def fire_gather(c, b):
        for k in range(SUPER):
            pltpu.async_copy(
                table_hbm.at[idx_v.at[c * SUPER + k]],
                bufs[b].at[k], gsems[b])

    def wait_gather(c, b):
        for k in range(SUPER):
            pltpu.make_async_copy(
                table_hbm.at[idx_v.at[0]], bufs[b].at[k], gsems[b]).wait()

    def fire_scatter(c, b):
        pltpu.async_copy(
            bufs[b], out_hbm.at[pl.ds(base + c * SUPER, SUPER)], ssems[b])

    def wait_scatter(b):
        pltpu.make_async_copy(
            bufs[b], out_hbm.at[pl.ds(base, SUPER)], ssems[b]).wait()

    for b in range(LOOK):
        fire_gather(b, b)

    wait_gather(0, 0)
    fire_scatter(0, 0)
    fire_gather(LOOK, LOOK % NBUF)

    def step(c, b, tb):
        wait_gather(c, b)
        fire_scatter(c, b)
        wait_scatter(tb)
        fire_gather(c + LOOK, tb)

    nsteady = NSUP - LOOK - 1
    ngroups = nsteady // NBUF
    nrem = nsteady % NBUF

    def outer(g, carry):
        for bp in range(NBUF):
            c = g * NBUF + 1 + bp
            step(c, (bp + 1) % NBUF, (1 + bp + LOOK) % NBUF)
        return carry

    lax.fori_loop(0, ngroups, outer, 0)
    for r in range(nrem):
        c = ngroups * NBUF + 1 + r
        step(c, c % NBUF, (c + LOOK) % NBUF)

    for c in range(NSUP - LOOK, NSUP):
        b = c % NBUF
        wait_gather(c, b)
        fire_scatter(c, b)
    for b in range(NBUF):
        wait_scatter(b)


@jax.jit
def _emb_call(idx, weight):
    mesh = plsc.VectorSubcoreMesh(
        core_axis_name="c", subcore_axis_name="s",
        num_cores=NUM_CORES, num_subcores=NUM_SUBCORES,
    )
    run = pl.kernel(
        _emb_body,
        out_type=jax.ShapeDtypeStruct((BATCH, HIST, EMB_DIM), jnp.float32),
        mesh=mesh,
        scratch_types=(
            [pltpu.VMEM((BATCH_PER_WORKER, HIST), jnp.int32)]
            + [pltpu.VMEM((SUPER, HIST, EMB_DIM), jnp.float32) for _ in range(NBUF)]
            + [pltpu.SemaphoreType.DMA for _ in range(2 * NBUF)]
        ),
    )
    return run(idx, weight)


def kernel(input, weight):
    idx = input.astype(jnp.int32).reshape(NUM_WORKERS, BATCH_PER_WORKER, HIST)
    return _emb_call(idx, weight)


# speedup vs baseline: 1.7890x; 1.0011x over previous
import jax
import jax.numpy as jnp
from jax import lax
from jax.experimental import pallas as pl
from jax.experimental.pallas import tpu as pltpu
from jax.experimental.pallas import tpu_sc as plsc

VOCAB = 100000
EMB_DIM = 128
BATCH = 4096
HIST = 50

NUM_CORES = 2
NUM_SUBCORES = 16
NUM_WORKERS = NUM_CORES * NUM_SUBCORES
BATCH_PER_WORKER = BATCH // NUM_WORKERS
SUPER = 4
NSUP = BATCH_PER_WORKER // SUPER
NBUF = 4
LOOK = NBUF - 1


def _emb_body(idx_hbm, table_hbm, out_hbm, idx_v, *bufs_and_sems):
    bufs = bufs_and_sems[:NBUF]
    gsems = bufs_and_sems[NBUF:2 * NBUF]
    ssems = bufs_and_sems[2 * NBUF:3 * NBUF]

    wid = lax.axis_index("s") * NUM_CORES + lax.axis_index("c")
    pltpu.sync_copy(idx_hbm.at[wid], idx_v)
    base = wid * BATCH_PER_WORKER

    def fire_gather(c, b):
        for k in range(SUPER):
            pltpu.async_copy(
                table_hbm.at[idx_v.at[c * SUPER + k]],
                bufs[b].at[k], gsems[b])

    def wait_gather(c, b):
        for k in range(SUPER):
            pltpu.make_async_copy(
                table_hbm.at[idx_v.at[0]], bufs[b].at[k], gsems[b]).wait()

    def fire_scatter(c, b):
        pltpu.async_copy(
            bufs[b], out_hbm.at[pl.ds(base + c * SUPER, SUPER)], ssems[b])

    def wait_scatter(b):
        pltpu.make_async_copy(
            bufs[b], out_hbm.at[pl.ds(base, SUPER)], ssems[b]).wait()

    for b in range(LOOK):
        fire_gather(b, b)

    wait_gather(0, 0)
    fire_scatter(0, 0)
    fire_gather(LOOK, LOOK % NBUF)

    def step(c, b, tb):
        wait_gather(c, b)
        fire_scatter(c, b)
        wait_scatter(tb)
        fire_gather(c + LOOK, tb)

    nsteady = NSUP - LOOK - 1
    ngroups = nsteady // NBUF
    nrem = nsteady % NBUF

    def outer(g, carry):
        for bp in range(NBUF):
            c = g * NBUF + 1 + bp
            step(c, (bp + 1) % NBUF, (1 + bp + LOOK) % NBUF)
        return carry

    lax.fori_loop(0, ngroups, outer, 0)
    for r in range(nrem):
        c = ngroups * NBUF + 1 + r
        step(c, c % NBUF, (c + LOOK) % NBUF)

    for c in range(NSUP - LOOK, NSUP):
        b = c % NBUF
        wait_gather(c, b)
        fire_scatter(c, b)
    for b in range(NBUF):
        wait_scatter(b)


@jax.jit
def _emb_call(idx, weight):
    mesh = plsc.VectorSubcoreMesh(
        core_axis_name="c", subcore_axis_name="s",
        num_cores=NUM_CORES, num_subcores=NUM_SUBCORES,
    )
    run = pl.kernel(
        _emb_body,
        out_type=jax.ShapeDtypeStruct((BATCH, HIST, EMB_DIM), jnp.float32),
        mesh=mesh,
        compiler_params=pltpu.CompilerParams(use_tc_tiling_on_sc=True),
        scratch_types=(
            [pltpu.VMEM((BATCH_PER_WORKER, HIST), jnp.int32)]
            + [pltpu.VMEM((SUPER, HIST, EMB_DIM), jnp.float32) for _ in range(NBUF)]
            + [pltpu.SemaphoreType.DMA for _ in range(2 * NBUF)]
        ),
    )
    return run(idx, weight)


def kernel(input, weight):
    idx = input.astype(jnp.int32).reshape(NUM_WORKERS, BATCH_PER_WORKER, HIST)
    return _emb_call(idx, weight)


# speedup vs baseline: 1.7901x; 1.0006x over previous
import jax
import jax.numpy as jnp
from jax import lax
from jax.experimental import pallas as pl
from jax.experimental.pallas import tpu as pltpu
from jax.experimental.pallas import tpu_sc as plsc

VOCAB = 100000
EMB_DIM = 128
BATCH = 4096
HIST = 50

NUM_CORES = 2
NUM_SUBCORES = 16
NUM_WORKERS = NUM_CORES * NUM_SUBCORES
BATCH_PER_WORKER = BATCH // NUM_WORKERS
SUPER = 4
NSUP = BATCH_PER_WORKER // SUPER
NBUF = 4
LOOK = NBUF - 1


def _emb_body(idx_hbm, table_hbm, out_hbm, idx_v, *bufs_and_sems):
    bufs = bufs_and_sems[:NBUF]
    gsems = bufs_and_sems[NBUF:2 * NBUF]
    ssems = bufs_and_sems[2 * NBUF:3 * NBUF]

    wid = lax.axis_index("s") * NUM_CORES + lax.axis_index("c")
    pltpu.sync_copy(idx_hbm.at[wid], idx_v)
    base = wid * BATCH_PER_WORKER

    def fire_gather(c, b):
        for k in range(SUPER):
            pltpu.async_copy(
                table_hbm.at[idx_v.at[c * SUPER + k]],
                bufs[b].at[k], gsems[b])

    def wait_gather(c, b):
        for k in range(SUPER):
            pltpu.make_async_copy(
                table_hbm.at[idx_v.at[0]], bufs[b].at[k], gsems[b]).wait()

    def fire_scatter(c, b):
        pltpu.async_copy(
            bufs[b], out_hbm.at[pl.ds(base + c * SUPER, SUPER)], ssems[b])

    def wait_scatter(b):
        pltpu.make_async_copy(
            bufs[b], out_hbm.at[pl.ds(base, SUPER)], ssems[b]).wait()

    for b in range(LOOK):
        fire_gather(b, b)

    wait_gather(0, 0)
    fire_scatter(0, 0)
    fire_gather(LOOK, LOOK % NBUF)

    def step(c, b, tb):
        wait_gather(c, b)
        fire_scatter(c, b)
        wait_scatter(tb)
        fire_gather(c + LOOK, tb)

    nsteady = NSUP - LOOK - 1
    ngroups = nsteady // NBUF
    nrem = nsteady % NBUF

    def outer(g, carry):
        for bp in range(NBUF):
            c = g * NBUF + 1 + bp
            step(c, (bp + 1) % NBUF, (1 + bp + LOOK) % NBUF)
        return carry

    lax.fori_loop(0, ngroups, outer, 0)
    for r in range(nrem):
        c = ngroups * NBUF + 1 + r
        step(c, c % NBUF, (c + LOOK) % NBUF)

    for c in range(NSUP - LOOK, NSUP):
        b = c % NBUF
        wait_gather(c, b)
        fire_scatter(c, b)
    for b in range(NBUF):
        wait_scatter(b)


@jax.jit
def _emb_call(idx, weight):
    mesh = plsc.VectorSubcoreMesh(
        core_axis_name="c", subcore_axis_name="s",
        num_cores=NUM_CORES, num_subcores=NUM_SUBCORES,
    )
    run = pl.kernel(
        _emb_body,
        out_type=jax.ShapeDtypeStruct((BATCH, HIST, EMB_DIM), jnp.float32),
        mesh=mesh,
        compiler_params=pltpu.CompilerParams(needs_layout_passes=True),
        scratch_types=(
            [pltpu.VMEM((BATCH_PER_WORKER, HIST), jnp.int32)]
            + [pltpu.VMEM((SUPER, HIST, EMB_DIM), jnp.float32) for _ in range(NBUF)]
            + [pltpu.SemaphoreType.DMA for _ in range(2 * NBUF)]
        ),
    )
    return run(idx, weight)


def kernel(input, weight):
    idx = input.astype(jnp.int32).reshape(NUM_WORKERS, BATCH_PER_WORKER, HIST)
    return _emb_call(idx, weight)
